# baseline (speedup 1.0000x reference)
import jax
import jax.numpy as jnp
from jax import lax
from jax.experimental import pallas as pl
from jax.experimental.pallas import tpu as pltpu
from jax.experimental.pallas import tpu_sc as plsc

B, L, D = 16, 4096, 768
NSEG = 128
LANES = 16
DH = D // 2
JV = DH // LANES
CHUNK = 64
NCHUNK = L // CHUNK


def _body(x_hbm, idx_hbm, out_hbm, xb0, xb1, ib, acc, cnt, sem0, sem1):
    c = lax.axis_index("c")
    s = lax.axis_index("s")
    b = c * 8 + s % 8
    dh = s // 8

    zero16 = jnp.zeros((LANES,), jnp.float32)
    one16 = jnp.ones((LANES,), jnp.float32)

    col0 = dh * DH

    def xsrc(ch):
        return x_hbm.at[b, pl.ds(ch * CHUNK, CHUNK), pl.ds(col0, DH)]

    pltpu.make_async_copy(xsrc(0), xb0, sem0).start()

    def zero_row(r, _):
        for j in range(JV):
            acc[r, pl.ds(j * LANES, LANES)] = zero16
        cnt[r, pl.ds(0, LANES)] = zero16
        return 0
    lax.fori_loop(0, NSEG, zero_row, 0)

    pltpu.sync_copy(idx_hbm.at[b, :], ib)

    def flush(cur_seg, run_len, accs):
        @pl.when(cur_seg >= 0)
        def _():
            rl = run_len.astype(jnp.float32)
            plsc.addupdate(cnt.at[cur_seg, pl.ds(0, LANES)],
                           jnp.broadcast_to(rl, (LANES,)))
            for j in range(JV):
                plsc.addupdate(acc.at[cur_seg, pl.ds(j * LANES, LANES)],
                               accs[j])

    def token_step(carry, xb, row, seg):
        cur_seg, run_len, accs = carry
        changed = seg != cur_seg

        @pl.when(changed)
        def _():
            flush(cur_seg, run_len, accs)

        vs = [xb[row, pl.ds(j * LANES, LANES)] for j in range(JV)]
        keep = jnp.broadcast_to(~changed, (LANES,))
        accs2 = tuple(jnp.where(keep, accs[j], 0.0) + vs[j]
                      for j in range(JV))
        run_len2 = jnp.where(changed, 1, run_len + 1)
        return (seg, run_len2, accs2)

    iota16 = lax.iota(jnp.int32, LANES)

    def process(xb, ch, carry):
        def grp(g, carry):
            t0 = ch * CHUNK + g * LANES
            ids = ib[pl.ds(t0, LANES)]
            cur_seg = carry[0]
            sh = ids.at[jnp.maximum(iota16 - 1, 0)].get(
                mode="promise_in_bounds")
            prev = jnp.where(iota16 == 0, cur_seg, sh)
            nb = plsc.all_reduce_population_count(ids != prev)[0]

            def fast(carry):
                cur_seg, run_len, accs = carry
                accs = list(accs)
                for l in range(LANES):
                    vs = [xb[g * LANES + l, pl.ds(j * LANES, LANES)]
                          for j in range(JV)]
                    for j in range(JV):
                        accs[j] = accs[j] + vs[j]
                return (cur_seg, run_len + LANES, tuple(accs))

            def slow(carry):
                for l in range(LANES):
                    carry = token_step(carry, xb, g * LANES + l, ids[l])
                return carry

            return lax.cond(nb == 0, fast, slow, carry)
        return lax.fori_loop(0, CHUNK // LANES, grp, carry)

    def pair_body(p, carry):
        ch0 = 2 * p
        pltpu.make_async_copy(xsrc(ch0 + 1), xb1, sem1).start()
        pltpu.make_async_copy(xsrc(ch0), xb0, sem0).wait()
        carry = process(xb0, ch0, carry)
        @pl.when(p < NCHUNK // 2 - 1)
        def _():
            pltpu.make_async_copy(xsrc(ch0 + 2), xb0, sem0).start()
        pltpu.make_async_copy(xsrc(ch0 + 1), xb1, sem1).wait()
        carry = process(xb1, ch0 + 1, carry)
        return carry

    zero_accs = tuple(zero16 for _ in range(JV))
    init = (jnp.int32(-1), jnp.int32(0), zero_accs)
    cur_seg, run_len, accs = lax.fori_loop(0, NCHUNK // 2, pair_body, init)
    flush(cur_seg, run_len, accs)

    def fin_row(r, _):
        rinv = 1.0 / jnp.maximum(cnt[r, pl.ds(0, LANES)], 1.0)
        for j in range(JV):
            acc[r, pl.ds(j * LANES, LANES)] = (
                acc[r, pl.ds(j * LANES, LANES)] * rinv)
        return 0
    lax.fori_loop(0, NSEG, fin_row, 0)

    pltpu.sync_copy(acc, out_hbm.at[b, :, pl.ds(col0, DH)])


def kernel(x, sentence_idx):
    mesh = plsc.VectorSubcoreMesh(core_axis_name="c", subcore_axis_name="s")
    f = pl.kernel(
        _body,
        out_type=jax.ShapeDtypeStruct((B, NSEG, D), jnp.float32),
        mesh=mesh,
        compiler_params=pltpu.CompilerParams(needs_layout_passes=False),
        scratch_types=[
            pltpu.VMEM((CHUNK, DH), jnp.float32),
            pltpu.VMEM((CHUNK, DH), jnp.float32),
            pltpu.VMEM((L,), jnp.int32),
            pltpu.VMEM((NSEG, DH), jnp.float32),
            pltpu.VMEM((NSEG, LANES), jnp.float32),
            pltpu.SemaphoreType.DMA,
            pltpu.SemaphoreType.DMA,
        ],
    )
    return f(x, sentence_idx.astype(jnp.int32))


# speedup vs baseline: 2.9436x; 2.9436x over previous
import jax
import jax.numpy as jnp
from jax import lax
from jax.experimental import pallas as pl
from jax.experimental.pallas import tpu as pltpu
from jax.experimental.pallas import tpu_sc as plsc

B, L, D = 16, 4096, 768
NSEG = 128
LANES = 16
DH = D // 2
JV = DH // LANES
CHUNK = 64
NCHUNK = L // CHUNK


def _body(x_hbm, idx_hbm, out_hbm, xb0, xb1, ib, acc, cnt, sem0, sem1,
          sm_bnd, sm_seg):
    c = lax.axis_index("c")
    s = lax.axis_index("s")
    b = c * 8 + s % 8
    dh = s // 8

    zero16 = jnp.zeros((LANES,), jnp.float32)
    one16 = jnp.ones((LANES,), jnp.float32)

    col0 = dh * DH

    def xsrc(ch):
        return x_hbm.at[b, pl.ds(ch * CHUNK, CHUNK), pl.ds(col0, DH)]

    pltpu.make_async_copy(xsrc(0), xb0, sem0).start()

    def zero_row(r, _):
        for j in range(JV):
            acc[r, pl.ds(j * LANES, LANES)] = zero16
        cnt[r, pl.ds(0, LANES)] = zero16
        return 0
    lax.fori_loop(0, NSEG, zero_row, 0)

    pltpu.sync_copy(idx_hbm.at[b, :], ib)

    def flush(cur_seg, run_len, accs):
        @pl.when(cur_seg >= 0)
        def _():
            rl = run_len.astype(jnp.float32)
            plsc.addupdate(cnt.at[cur_seg, pl.ds(0, LANES)],
                           jnp.broadcast_to(rl, (LANES,)))
            for j in range(JV):
                plsc.addupdate(acc.at[cur_seg, pl.ds(j * LANES, LANES)],
                               accs[j])

    iota16 = lax.iota(jnp.int32, LANES)

    def scan_runs(ch, prev_last):
        c0 = ch * CHUNK
        sm_bnd[0] = 0
        nb = jnp.int32(1)
        prev = prev_last
        for g in range(CHUNK // LANES):
            idv = ib[pl.ds(c0 + g * LANES, LANES)]
            sh = idv.at[jnp.maximum(iota16 - 1, 0)].get(
                mode="promise_in_bounds")
            pv = jnp.where(iota16 == 0, prev, sh)
            chg = (idv != pv).astype(jnp.int32)
            for l in range(LANES):
                t = g * LANES + l
                if t == 0:
                    sm_seg[0] = idv[0]
                else:
                    cl = chg[l]
                    vl = idv[l]

                    @pl.when(cl == 1)
                    def _(nb=nb, vl=vl, t=t):
                        sm_bnd[nb] = t
                        sm_seg[nb] = vl
                    nb = nb + cl
            prev = idv[LANES - 1]
        sm_bnd[nb] = CHUNK
        return nb, prev

    def process(xb, ch, carry):
        cur_seg, run_len, accs, prev_last = carry
        nruns, prev_last = scan_runs(ch, prev_last)

        def run_body(k, rc):
            cur_seg, run_len, accs = rc
            st = sm_bnd[k]
            en = sm_bnd[k + 1]
            seg = sm_seg[k]
            changed = seg != cur_seg

            @pl.when(changed)
            def _():
                flush(cur_seg, run_len, accs)

            keep = jnp.broadcast_to(~changed, (LANES,))
            accs = tuple(jnp.where(keep, a, 0.0) for a in accs)
            run_len = jnp.where(changed, 0, run_len)

            def tok(t, accs):
                return tuple(accs[j] + xb[t, pl.ds(j * LANES, LANES)]
                             for j in range(JV))
            accs = lax.fori_loop(st, en, tok, accs)
            return (seg, run_len + (en - st), accs)

        rc = lax.fori_loop(0, nruns, run_body, (cur_seg, run_len, accs))
        return (*rc, prev_last)

    def pair_body(p, carry):
        ch0 = 2 * p
        pltpu.make_async_copy(xsrc(ch0 + 1), xb1, sem1).start()
        pltpu.make_async_copy(xsrc(ch0), xb0, sem0).wait()
        carry = process(xb0, ch0, carry)
        @pl.when(p < NCHUNK // 2 - 1)
        def _():
            pltpu.make_async_copy(xsrc(ch0 + 2), xb0, sem0).start()
        pltpu.make_async_copy(xsrc(ch0 + 1), xb1, sem1).wait()
        carry = process(xb1, ch0 + 1, carry)
        return carry

    zero_accs = tuple(zero16 for _ in range(JV))
    init = (jnp.int32(-1), jnp.int32(0), zero_accs, jnp.int32(-1))
    cur_seg, run_len, accs, _ = lax.fori_loop(0, NCHUNK // 2, pair_body, init)
    flush(cur_seg, run_len, accs)

    def fin_row(r, _):
        rinv = 1.0 / jnp.maximum(cnt[r, pl.ds(0, LANES)], 1.0)
        for j in range(JV):
            acc[r, pl.ds(j * LANES, LANES)] = (
                acc[r, pl.ds(j * LANES, LANES)] * rinv)
        return 0
    lax.fori_loop(0, NSEG, fin_row, 0)

    pltpu.sync_copy(acc, out_hbm.at[b, :, pl.ds(col0, DH)])


def kernel(x, sentence_idx):
    mesh = plsc.VectorSubcoreMesh(core_axis_name="c", subcore_axis_name="s")
    f = pl.kernel(
        _body,
        out_type=jax.ShapeDtypeStruct((B, NSEG, D), jnp.float32),
        mesh=mesh,
        compiler_params=pltpu.CompilerParams(needs_layout_passes=False),
        scratch_types=[
            pltpu.VMEM((CHUNK, DH), jnp.float32),
            pltpu.VMEM((CHUNK, DH), jnp.float32),
            pltpu.VMEM((L,), jnp.int32),
            pltpu.VMEM((NSEG, DH), jnp.float32),
            pltpu.VMEM((NSEG, LANES), jnp.float32),
            pltpu.SemaphoreType.DMA,
            pltpu.SemaphoreType.DMA,
            pltpu.SMEM((CHUNK + 1,), jnp.int32),
            pltpu.SMEM((CHUNK,), jnp.int32),
        ],
    )
    return f(x, sentence_idx.astype(jnp.int32))


# speedup vs baseline: 4.1884x; 1.4229x over previous
import jax
import jax.numpy as jnp
from jax import lax
from jax.experimental import pallas as pl
from jax.experimental.pallas import tpu as pltpu
from jax.experimental.pallas import tpu_sc as plsc

B, L, D = 16, 4096, 768
NSEG = 128
LANES = 16
DH = D // 2
JV = DH // LANES
CHUNK = 64
NCHUNK = L // CHUNK


def _body(x_hbm, idx_hbm, out_hbm, xb0, xb1, ib, acc, cnt, sem0, sem1,
          sm_bnd, sm_seg, bndv, segv):
    c = lax.axis_index("c")
    s = lax.axis_index("s")
    b = c * 8 + s % 8
    dh = s // 8

    zero16 = jnp.zeros((LANES,), jnp.float32)
    one16 = jnp.ones((LANES,), jnp.float32)

    col0 = dh * DH

    def xsrc(ch):
        return x_hbm.at[b, pl.ds(ch * CHUNK, CHUNK), pl.ds(col0, DH)]

    pltpu.make_async_copy(xsrc(0), xb0, sem0).start()

    def zero_row(r, _):
        for j in range(JV):
            acc[r, pl.ds(j * LANES, LANES)] = zero16
        cnt[r, pl.ds(0, LANES)] = zero16
        return 0
    lax.fori_loop(0, NSEG, zero_row, 0)

    pltpu.sync_copy(idx_hbm.at[b, :], ib)

    def flush(cur_seg, run_len, accs):
        @pl.when(cur_seg >= 0)
        def _():
            rl = run_len.astype(jnp.float32)
            plsc.addupdate(cnt.at[cur_seg, pl.ds(0, LANES)],
                           jnp.broadcast_to(rl, (LANES,)))
            for j in range(JV):
                plsc.addupdate(acc.at[cur_seg, pl.ds(j * LANES, LANES)],
                               accs[j])

    iota16 = lax.iota(jnp.int32, LANES)

    def scan_runs(ch, prev_last):
        c0 = ch * CHUNK
        nb = jnp.int32(0)
        prev = prev_last
        for g in range(CHUNK // LANES):
            idv = ib[pl.ds(c0 + g * LANES, LANES)]
            sh = idv.at[jnp.maximum(iota16 - 1, 0)].get(
                mode="promise_in_bounds")
            pv = jnp.where(iota16 == 0, prev, sh)
            chg = idv != pv
            if g == 0:
                chg = chg | (iota16 == 0)
            pos = plsc.cumsum(chg.astype(jnp.int32))
            slot = pos + (nb - 1)
            plsc.store_scatter(bndv, [slot], iota16 + g * LANES, mask=chg)
            plsc.store_scatter(segv, [slot], idv, mask=chg)
            nb = nb + pos[LANES - 1]
            prev = idv[LANES - 1]
        plsc.store_scatter(bndv, [jnp.broadcast_to(nb, (LANES,))],
                           jnp.full((LANES,), CHUNK, jnp.int32),
                           mask=iota16 == 0)

        def batch(bi, _):
            off = pl.multiple_of(bi * LANES, LANES)
            bv = bndv[pl.ds(off, LANES)]
            sv = segv[pl.ds(off, LANES)]
            for l in range(LANES):
                sm_bnd[off + l] = bv[l]
                sm_seg[off + l] = sv[l]
            return 0
        lax.fori_loop(0, (nb + LANES) // LANES, batch, 0)
        return nb, prev

    def process(xb, ch, carry):
        cur_seg, run_len, accs, prev_last = carry
        nruns, prev_last = scan_runs(ch, prev_last)

        def run_body(k, rc):
            cur_seg, run_len, accs = rc
            st = sm_bnd[k]
            en = sm_bnd[k + 1]
            seg = sm_seg[k]
            changed = seg != cur_seg

            @pl.when(changed)
            def _():
                flush(cur_seg, run_len, accs)

            keep = jnp.broadcast_to(~changed, (LANES,))
            accs = tuple(jnp.where(keep, a, 0.0) for a in accs)
            run_len = jnp.where(changed, 0, run_len)

            def tok(t, accs):
                return tuple(accs[j] + xb[t, pl.ds(j * LANES, LANES)]
                             for j in range(JV))
            accs = lax.fori_loop(st, en, tok, accs)
            return (seg, run_len + (en - st), accs)

        rc = lax.fori_loop(0, nruns, run_body, (cur_seg, run_len, accs))
        return (*rc, prev_last)

    def pair_body(p, carry):
        ch0 = 2 * p
        pltpu.make_async_copy(xsrc(ch0 + 1), xb1, sem1).start()
        pltpu.make_async_copy(xsrc(ch0), xb0, sem0).wait()
        carry = process(xb0, ch0, carry)
        @pl.when(p < NCHUNK // 2 - 1)
        def _():
            pltpu.make_async_copy(xsrc(ch0 + 2), xb0, sem0).start()
        pltpu.make_async_copy(xsrc(ch0 + 1), xb1, sem1).wait()
        carry = process(xb1, ch0 + 1, carry)
        return carry

    zero_accs = tuple(zero16 for _ in range(JV))
    init = (jnp.int32(-1), jnp.int32(0), zero_accs, jnp.int32(-1))
    cur_seg, run_len, accs, _ = lax.fori_loop(0, NCHUNK // 2, pair_body, init)
    flush(cur_seg, run_len, accs)

    def fin_row(r, _):
        rinv = 1.0 / jnp.maximum(cnt[r, pl.ds(0, LANES)], 1.0)
        for j in range(JV):
            acc[r, pl.ds(j * LANES, LANES)] = (
                acc[r, pl.ds(j * LANES, LANES)] * rinv)
        return 0
    lax.fori_loop(0, NSEG, fin_row, 0)

    pltpu.sync_copy(acc, out_hbm.at[b, :, pl.ds(col0, DH)])


def kernel(x, sentence_idx):
    mesh = plsc.VectorSubcoreMesh(core_axis_name="c", subcore_axis_name="s")
    f = pl.kernel(
        _body,
        out_type=jax.ShapeDtypeStruct((B, NSEG, D), jnp.float32),
        mesh=mesh,
        compiler_params=pltpu.CompilerParams(needs_layout_passes=False),
        scratch_types=[
            pltpu.VMEM((CHUNK, DH), jnp.float32),
            pltpu.VMEM((CHUNK, DH), jnp.float32),
            pltpu.VMEM((L,), jnp.int32),
            pltpu.VMEM((NSEG, DH), jnp.float32),
            pltpu.VMEM((NSEG, LANES), jnp.float32),
            pltpu.SemaphoreType.DMA,
            pltpu.SemaphoreType.DMA,
            pltpu.SMEM((CHUNK + LANES,), jnp.int32),
            pltpu.SMEM((CHUNK + LANES,), jnp.int32),
            pltpu.VMEM((CHUNK + LANES,), jnp.int32),
            pltpu.VMEM((CHUNK + LANES,), jnp.int32),
        ],
    )
    return f(x, sentence_idx.astype(jnp.int32))


# speedup vs baseline: 4.2382x; 1.0119x over previous
import jax
import jax.numpy as jnp
from jax import lax
from jax.experimental import pallas as pl
from jax.experimental.pallas import tpu as pltpu
from jax.experimental.pallas import tpu_sc as plsc

B, L, D = 16, 4096, 768
NSEG = 128
LANES = 16
DH = D // 2
JV = DH // LANES
CHUNK = 64
NCHUNK = L // CHUNK


def _body(x_hbm, idx_hbm, out_hbm, xb0, xb1, ib, acc, cnt, sem0, sem1, semi,
          sm_bnd, sm_seg, bndv, segv):
    c = lax.axis_index("c")
    s = lax.axis_index("s")
    b = c * 8 + s % 8
    dh = s // 8

    zero16 = jnp.zeros((LANES,), jnp.float32)
    one16 = jnp.ones((LANES,), jnp.float32)

    col0 = dh * DH

    def xsrc(ch):
        return x_hbm.at[b, pl.ds(ch * CHUNK, CHUNK), pl.ds(col0, DH)]

    pltpu.make_async_copy(xsrc(0), xb0, sem0).start()
    idx_cp = pltpu.make_async_copy(idx_hbm.at[b, :], ib, semi)
    idx_cp.start()

    def zero_row(r, _):
        for j in range(JV):
            acc[r, pl.ds(j * LANES, LANES)] = zero16
        cnt[r, pl.ds(0, LANES)] = zero16
        return 0
    lax.fori_loop(0, NSEG, zero_row, 0)

    idx_cp.wait()

    def flush(cur_seg, run_len, accs):
        @pl.when(cur_seg >= 0)
        def _():
            rl = run_len.astype(jnp.float32)
            plsc.addupdate(cnt.at[cur_seg, pl.ds(0, LANES)],
                           jnp.broadcast_to(rl, (LANES,)))
            for j in range(JV):
                plsc.addupdate(acc.at[cur_seg, pl.ds(j * LANES, LANES)],
                               accs[j])

    iota16 = lax.iota(jnp.int32, LANES)

    def scan_runs(ch, prev_last):
        c0 = ch * CHUNK
        nb = jnp.int32(0)
        prev = prev_last
        for g in range(CHUNK // LANES):
            idv = ib[pl.ds(c0 + g * LANES, LANES)]
            sh = idv.at[jnp.maximum(iota16 - 1, 0)].get(
                mode="promise_in_bounds")
            pv = jnp.where(iota16 == 0, prev, sh)
            chg = idv != pv
            if g == 0:
                chg = chg | (iota16 == 0)
            pos = plsc.cumsum(chg.astype(jnp.int32))
            slot = pos + (nb - 1)
            plsc.store_scatter(bndv, [slot], iota16 + g * LANES, mask=chg)
            plsc.store_scatter(segv, [slot], idv, mask=chg)
            nb = nb + pos[LANES - 1]
            prev = idv[LANES - 1]
        plsc.store_scatter(bndv, [jnp.broadcast_to(nb, (LANES,))],
                           jnp.full((LANES,), CHUNK, jnp.int32),
                           mask=iota16 == 0)

        def batch(bi, _):
            off = pl.multiple_of(bi * LANES, LANES)
            bv = bndv[pl.ds(off, LANES)]
            sv = segv[pl.ds(off, LANES)]
            for l in range(LANES):
                sm_bnd[off + l] = bv[l]
                sm_seg[off + l] = sv[l]
            return 0
        lax.fori_loop(0, (nb + LANES) // LANES, batch, 0)
        return nb, prev

    def process(xb, ch, carry):
        cur_seg, run_len, accs, prev_last = carry
        nruns, prev_last = scan_runs(ch, prev_last)

        def run_body(k, rc):
            cur_seg, run_len, accs = rc
            st = sm_bnd[k]
            en = sm_bnd[k + 1]
            seg = sm_seg[k]
            changed = seg != cur_seg

            @pl.when(changed)
            def _():
                flush(cur_seg, run_len, accs)

            keep = jnp.broadcast_to(~changed, (LANES,))
            accs = tuple(jnp.where(keep, a, 0.0) for a in accs)
            run_len = jnp.where(changed, 0, run_len)

            def tok(t, accs):
                return tuple(accs[j] + xb[t, pl.ds(j * LANES, LANES)]
                             for j in range(JV))
            accs = lax.fori_loop(st, en, tok, accs)
            return (seg, run_len + (en - st), accs)

        rc = lax.fori_loop(0, nruns, run_body, (cur_seg, run_len, accs))
        return (*rc, prev_last)

    def pair_body(p, carry):
        ch0 = 2 * p
        pltpu.make_async_copy(xsrc(ch0 + 1), xb1, sem1).start()
        pltpu.make_async_copy(xsrc(ch0), xb0, sem0).wait()
        carry = process(xb0, ch0, carry)
        @pl.when(p < NCHUNK // 2 - 1)
        def _():
            pltpu.make_async_copy(xsrc(ch0 + 2), xb0, sem0).start()
        pltpu.make_async_copy(xsrc(ch0 + 1), xb1, sem1).wait()
        carry = process(xb1, ch0 + 1, carry)
        return carry

    zero_accs = tuple(zero16 for _ in range(JV))
    init = (jnp.int32(-1), jnp.int32(0), zero_accs, jnp.int32(-1))
    cur_seg, run_len, accs, _ = lax.fori_loop(0, NCHUNK // 2, pair_body, init)
    flush(cur_seg, run_len, accs)

    def fin_row(r, _):
        rinv = 1.0 / jnp.maximum(cnt[r, pl.ds(0, LANES)], 1.0)
        for j in range(JV):
            acc[r, pl.ds(j * LANES, LANES)] = (
                acc[r, pl.ds(j * LANES, LANES)] * rinv)
        return 0

    lax.fori_loop(0, NSEG // 2, fin_row, 0)
    out_cp = pltpu.make_async_copy(
        acc.at[pl.ds(0, NSEG // 2), :],
        out_hbm.at[b, pl.ds(0, NSEG // 2), pl.ds(col0, DH)], semi)
    out_cp.start()
    lax.fori_loop(NSEG // 2, NSEG, fin_row, 0)
    pltpu.sync_copy(acc.at[pl.ds(NSEG // 2, NSEG // 2), :],
                    out_hbm.at[b, pl.ds(NSEG // 2, NSEG // 2),
                               pl.ds(col0, DH)])
    out_cp.wait()


def kernel(x, sentence_idx):
    mesh = plsc.VectorSubcoreMesh(core_axis_name="c", subcore_axis_name="s")
    f = pl.kernel(
        _body,
        out_type=jax.ShapeDtypeStruct((B, NSEG, D), jnp.float32),
        mesh=mesh,
        compiler_params=pltpu.CompilerParams(needs_layout_passes=False),
        scratch_types=[
            pltpu.VMEM((CHUNK, DH), jnp.float32),
            pltpu.VMEM((CHUNK, DH), jnp.float32),
            pltpu.VMEM((L,), jnp.int32),
            pltpu.VMEM((NSEG, DH), jnp.float32),
            pltpu.VMEM((NSEG, LANES), jnp.float32),
            pltpu.SemaphoreType.DMA,
            pltpu.SemaphoreType.DMA,
            pltpu.SemaphoreType.DMA,
            pltpu.SMEM((CHUNK + LANES,), jnp.int32),
            pltpu.SMEM((CHUNK + LANES,), jnp.int32),
            pltpu.VMEM((CHUNK + LANES,), jnp.int32),
            pltpu.VMEM((CHUNK + LANES,), jnp.int32),
        ],
    )
    return f(x, sentence_idx.astype(jnp.int32))


# speedup vs baseline: 4.2443x; 1.0014x over previous
import jax
import jax.numpy as jnp
from jax import lax
from jax.experimental import pallas as pl
from jax.experimental.pallas import tpu as pltpu
from jax.experimental.pallas import tpu_sc as plsc

B, L, D = 16, 4096, 768
NSEG = 128
LANES = 16
DH = D // 2
JV = DH // LANES
CHUNK = 64
NCHUNK = L // CHUNK


def _body(x_hbm, idx_hbm, out_hbm, xb0, xb1, ib, acc, cnt, sem0, sem1, semi,
          sm_bnd, sm_seg, bndv, segv):
    c = lax.axis_index("c")
    s = lax.axis_index("s")
    b = c * 8 + s % 8
    dh = s // 8

    zero16 = jnp.zeros((LANES,), jnp.float32)
    col0 = dh * DH

    def xsrc(ch):
        return x_hbm.at[b, pl.ds(ch * CHUNK, CHUNK), pl.ds(col0, DH)]

    pltpu.make_async_copy(xsrc(0), xb0, sem0).start()
    idx_cp = pltpu.make_async_copy(idx_hbm.at[b, :], ib, semi)
    idx_cp.start()

    def zero_row(r, _):
        for j in range(JV):
            acc[r, pl.ds(j * LANES, LANES)] = zero16
        cnt[r, pl.ds(0, LANES)] = zero16
        return 0
    lax.fori_loop(0, NSEG, zero_row, 0)

    idx_cp.wait()

    def flush(cur_seg, run_len, accs):
        @pl.when(cur_seg >= 0)
        def _():
            rl = run_len.astype(jnp.float32)
            plsc.addupdate(cnt.at[cur_seg, pl.ds(0, LANES)],
                           jnp.broadcast_to(rl, (LANES,)))
            for j in range(JV):
                plsc.addupdate(acc.at[cur_seg, pl.ds(j * LANES, LANES)],
                               accs[j])

    iota16 = lax.iota(jnp.int32, LANES)

    def scan_runs(ch, prev_last):
        c0 = ch * CHUNK
        nb = jnp.int32(0)
        prev = prev_last
        for g in range(CHUNK // LANES):
            idv = ib[pl.ds(c0 + g * LANES, LANES)]
            sh = idv.at[jnp.maximum(iota16 - 1, 0)].get(
                mode="promise_in_bounds")
            pv = jnp.where(iota16 == 0, prev, sh)
            chg = idv != pv
            if g == 0:
                chg = chg | (iota16 == 0)
            pos = plsc.cumsum(chg.astype(jnp.int32))
            slot = pos + (nb - 1)
            plsc.store_scatter(bndv, [slot], iota16 + g * LANES, mask=chg)
            plsc.store_scatter(segv, [slot], idv, mask=chg)
            nb = nb + pos[LANES - 1]
            prev = idv[LANES - 1]
        plsc.store_scatter(bndv, [jnp.broadcast_to(nb, (LANES,))],
                           jnp.full((LANES,), CHUNK, jnp.int32),
                           mask=iota16 == 0)

        def batch(bi, _):
            off = pl.multiple_of(bi * LANES, LANES)
            bv = bndv[pl.ds(off, LANES)]
            sv = segv[pl.ds(off, LANES)]
            for l in range(LANES):
                sm_bnd[off + l] = bv[l]
                sm_seg[off + l] = sv[l]
            return 0
        lax.fori_loop(0, (nb + LANES) // LANES, batch, 0)
        return nb, prev

    def process(xb, ch, carry):
        cur_seg, run_len, accs, prev_last = carry
        nruns, prev_last = scan_runs(ch, prev_last)

        def run_body(k, rc):
            cur_seg, run_len, accs = rc
            st = sm_bnd[k]
            en = sm_bnd[k + 1]
            seg = sm_seg[k]
            changed = seg != cur_seg

            @pl.when(changed)
            def _():
                flush(cur_seg, run_len, accs)

            keep = jnp.broadcast_to(~changed, (LANES,))
            accs = tuple(jnp.where(keep, a, 0.0) for a in accs)
            run_len = jnp.where(changed, 0, run_len)

            def tok(t, accs):
                return tuple(accs[j] + xb[t, pl.ds(j * LANES, LANES)]
                             for j in range(JV))
            accs = lax.fori_loop(st, en, tok, accs)
            return (seg, run_len + (en - st), accs)

        rc = lax.fori_loop(0, nruns, run_body, (cur_seg, run_len, accs))
        return (*rc, prev_last)

    def pair_body(p, carry):
        ch0 = 2 * p
        pltpu.make_async_copy(xsrc(ch0 + 1), xb1, sem1).start()
        pltpu.make_async_copy(xsrc(ch0), xb0, sem0).wait()
        carry = process(xb0, ch0, carry)
        @pl.when(p < NCHUNK // 2 - 1)
        def _():
            pltpu.make_async_copy(xsrc(ch0 + 2), xb0, sem0).start()
        pltpu.make_async_copy(xsrc(ch0 + 1), xb1, sem1).wait()
        carry = process(xb1, ch0 + 1, carry)
        return carry

    zero_accs = tuple(zero16 for _ in range(JV))
    init = (jnp.int32(-1), jnp.int32(0), zero_accs, jnp.int32(-1))
    cur_seg, run_len, accs, _ = lax.fori_loop(0, NCHUNK // 2, pair_body, init)
    flush(cur_seg, run_len, accs)

    def fin_row(r, _):
        rinv = 1.0 / jnp.maximum(cnt[r, pl.ds(0, LANES)], 1.0)
        for j in range(JV):
            acc[r, pl.ds(j * LANES, LANES)] = (
                acc[r, pl.ds(j * LANES, LANES)] * rinv)
        return 0

    lax.fori_loop(0, NSEG // 2, fin_row, 0)
    out_cp = pltpu.make_async_copy(
        acc.at[pl.ds(0, NSEG // 2), :],
        out_hbm.at[b, pl.ds(0, NSEG // 2), pl.ds(col0, DH)], semi)
    out_cp.start()
    lax.fori_loop(NSEG // 2, NSEG, fin_row, 0)
    pltpu.sync_copy(acc.at[pl.ds(NSEG // 2, NSEG // 2), :],
                    out_hbm.at[b, pl.ds(NSEG // 2, NSEG // 2),
                               pl.ds(col0, DH)])
    out_cp.wait()


def kernel(x, sentence_idx):
    mesh = plsc.VectorSubcoreMesh(core_axis_name="c", subcore_axis_name="s")
    f = pl.kernel(
        _body,
        out_type=jax.ShapeDtypeStruct((B, NSEG, D), jnp.float32),
        mesh=mesh,
        compiler_params=pltpu.CompilerParams(needs_layout_passes=False),
        scratch_types=[
            pltpu.VMEM((CHUNK, DH), jnp.float32),
            pltpu.VMEM((CHUNK, DH), jnp.float32),
            pltpu.VMEM((L,), jnp.int32),
            pltpu.VMEM((NSEG, DH), jnp.float32),
            pltpu.VMEM((NSEG, LANES), jnp.float32),
            pltpu.SemaphoreType.DMA,
            pltpu.SemaphoreType.DMA,
            pltpu.SemaphoreType.DMA,
            pltpu.SMEM((CHUNK + LANES,), jnp.int32),
            pltpu.SMEM((CHUNK + LANES,), jnp.int32),
            pltpu.VMEM((CHUNK + LANES,), jnp.int32),
            pltpu.VMEM((CHUNK + LANES,), jnp.int32),
        ],
    )
    return f(x, sentence_idx.astype(jnp.int32))
